# baseline (speedup 1.0000x reference)
import jax
import jax.numpy as jnp
from jax import lax
from jax.experimental import pallas as pl
from jax.experimental.pallas import tpu as pltpu
from jax.experimental.pallas import tpu_sc as plsc

B, S = 2, 2048
DIM = 1024
INTER = 512
E = 8
K = 2
NG = 4
T = B * S
A_TOT = T * K
BLK = 128
G = A_TOT + E * BLK
NB = G // BLK
TB = 512
TBS = 512
NEG = -1e30

NC = 2
NS = 16
NWK = NC * NS
A_PER_W = A_TOT // NWK
CH = 32
NCH = A_PER_W // CH


def _router_body(lg_ref, er_ref, rr_ref, vv_ref, cnt_ref):
    i = pl.program_id(0)

    @pl.when(i == 0)
    def _():
        cnt_ref[...] = jnp.zeros((8, 8), jnp.float32)

    lane = lax.broadcasted_iota(jnp.int32, (TB, E), 1)
    lg = lg_ref[...]
    m = jnp.max(lg, axis=1, keepdims=True)
    ex = jnp.exp(lg - m)
    s = ex / jnp.sum(ex, axis=1, keepdims=True)

    row = lax.broadcasted_iota(jnp.int32, (E, E), 0)
    col = lax.broadcasted_iota(jnp.int32, (E, E), 1)
    gm = jnp.where((col < NG) & (row // 2 == col), 1.0, 0.0)
    gmt = jnp.where((row < NG) & (col // 2 == row), 1.0, 0.0)
    gs = jnp.dot(s, gm, preferred_element_type=jnp.float32,
                 precision=lax.Precision.HIGHEST)
    gsm = jnp.where(lane < NG, gs, NEG)

    g1 = jnp.max(gsm, axis=1, keepdims=True)
    i1 = jnp.min(jnp.where(gsm == g1, lane, 128), axis=1, keepdims=True)
    gsm2 = jnp.where(lane == i1, NEG, gsm)
    g2 = jnp.max(gsm2, axis=1, keepdims=True)
    i2 = jnp.min(jnp.where(gsm2 == g2, lane, 128), axis=1, keepdims=True)
    selg = jnp.where((lane == i1) | (lane == i2), 1.0, 0.0)
    sele = jnp.dot(selg, gmt, preferred_element_type=jnp.float32,
                   precision=lax.Precision.HIGHEST)

    sp = jnp.where(sele > 0.5, s, NEG)
    v1 = jnp.max(sp, axis=1, keepdims=True)
    e1 = jnp.min(jnp.where(sp == v1, lane, 128), axis=1, keepdims=True)
    sp2 = jnp.where(lane == e1, NEG, sp)
    v2 = jnp.max(sp2, axis=1, keepdims=True)
    e2 = jnp.min(jnp.where(sp2 == v2, lane, 128), axis=1, keepdims=True)

    oh1 = jnp.where(lane == e1, 1.0, 0.0)
    oh2 = jnp.where(lane == e2, 1.0, 0.0)
    h = oh1 + oh2
    rr = lax.broadcasted_iota(jnp.int32, (TB, TB), 0)
    cc = lax.broadcasted_iota(jnp.int32, (TB, TB), 1)
    lf = jnp.where(cc < rr, 1.0, 0.0)
    p = jnp.dot(lf, h, preferred_element_type=jnp.float32)
    cb = cnt_ref[0:1, :]
    r1 = jnp.sum((p + cb) * oh1, axis=1, keepdims=True)
    r2 = jnp.sum((p + cb) * oh2, axis=1, keepdims=True)
    cnt_ref[0:1, :] = cb + jnp.sum(h, axis=0, keepdims=True)

    er_ref[...] = jnp.concatenate([e1, e2], axis=1)
    rr_ref[...] = jnp.concatenate([r1.astype(jnp.int32),
                                   r2.astype(jnp.int32)], axis=1)
    vv_ref[...] = jnp.concatenate([v1, v2], axis=1)


def _gmm_body(be_ref, x_ref, w1_ref, b1_ref, w2_ref, b2_ref, w3_ref, b3_ref,
              o_ref):
    a = x_ref[...]
    h1 = jnp.dot(a, w1_ref[0], preferred_element_type=jnp.float32) + b1_ref[0]
    h2 = jnp.dot(a, w2_ref[0], preferred_element_type=jnp.float32) + b2_ref[0]
    h = h1 * h2
    hs = h * jax.nn.sigmoid(h)
    o_ref[...] = jnp.dot(hs, w3_ref[0],
                         preferred_element_type=jnp.float32) + b3_ref[0]


def _shared_body(x_ref, ws1_ref, bs1_ref, ws2_ref, bs2_ref, ws3_ref, bs3_ref,
                 z_ref):
    a = x_ref[...]
    h1 = jnp.dot(a, ws1_ref[...], preferred_element_type=jnp.float32) + bs1_ref[...]
    h2 = jnp.dot(a, ws2_ref[...], preferred_element_type=jnp.float32) + bs2_ref[...]
    h = h1 * h2
    hs = h * jax.nn.sigmoid(h)
    z_ref[...] = jnp.dot(hs, ws3_ref[...],
                         preferred_element_type=jnp.float32) + bs3_ref[...]


def _combine_body(z_ref, og_ref, vv_ref, y_ref):
    vv = vv_ref[...]
    v1 = vv[:, 0:1]
    v2 = vv[:, 1:2]
    og = og_ref[...]
    y_ref[...] = z_ref[...] + v1 * og[:, :DIM] + v2 * og[:, DIM:]


def _sc_dispatch_body(x_hbm, src_hbm, pos_hbm, xg_hbm, src_v, pos_v,
                      rows_a, rows_b, sga, sgb, ssa, ssb):
    wid = lax.axis_index("s") * NC + lax.axis_index("c")
    base = wid * NCH
    pltpu.sync_copy(src_hbm.at[pl.ds(base, NCH)], src_v)
    pltpu.sync_copy(pos_hbm.at[pl.ds(base, NCH)], pos_v)
    bufs = (rows_a, rows_b)
    gsems = (sga, sgb)
    ssems = (ssa, ssb)
    hg = pltpu.async_copy(x_hbm.at[src_v.at[0]], rows_a, sga)
    hs = [None] * NCH
    for j in range(NCH):
        hg.wait()
        if j + 1 < NCH and j >= 1:
            hs[j - 1].wait()
        hs[j] = pltpu.async_copy(bufs[j % 2], xg_hbm.at[pos_v.at[j]],
                                 ssems[j % 2])
        if j + 1 < NCH:
            hg = pltpu.async_copy(x_hbm.at[src_v.at[j + 1]],
                                  bufs[(j + 1) % 2], gsems[(j + 1) % 2])
    hs[NCH - 2].wait()
    hs[NCH - 1].wait()


def _sc_gather_body(og_hbm, pos_hbm, out_hbm, pos_v, rows_a, rows_b,
                    sga, sgb, ssa, ssb):
    wid = lax.axis_index("s") * NC + lax.axis_index("c")
    base = wid * NCH
    abase = wid * A_PER_W
    pltpu.sync_copy(pos_hbm.at[pl.ds(base, NCH)], pos_v)
    bufs = (rows_a, rows_b)
    gsems = (sga, sgb)
    ssems = (ssa, ssb)
    hg = pltpu.async_copy(og_hbm.at[pos_v.at[0]], rows_a, sga)
    hs = [None] * NCH
    for j in range(NCH):
        hg.wait()
        if j + 1 < NCH and j >= 1:
            hs[j - 1].wait()
        hs[j] = pltpu.async_copy(bufs[j % 2],
                                 out_hbm.at[pl.ds(abase + j * CH, CH)],
                                 ssems[j % 2])
        if j + 1 < NCH:
            hg = pltpu.async_copy(og_hbm.at[pos_v.at[j + 1]],
                                  bufs[(j + 1) % 2], gsems[(j + 1) % 2])
    hs[NCH - 2].wait()
    hs[NCH - 1].wait()


def _make_sc_kernels():
    mesh = plsc.VectorSubcoreMesh(core_axis_name="c", subcore_axis_name="s",
                                  num_cores=NC, num_subcores=NS)
    dispatch = pl.kernel(
        _sc_dispatch_body,
        out_type=jax.ShapeDtypeStruct((G, DIM), jnp.float32),
        mesh=mesh,
        scratch_types=[
            pltpu.VMEM((NCH, CH), jnp.int32),
            pltpu.VMEM((NCH, CH), jnp.int32),
            pltpu.VMEM((CH, DIM), jnp.float32),
            pltpu.VMEM((CH, DIM), jnp.float32),
            pltpu.SemaphoreType.DMA,
            pltpu.SemaphoreType.DMA,
            pltpu.SemaphoreType.DMA,
            pltpu.SemaphoreType.DMA,
        ],
    )
    gather = pl.kernel(
        _sc_gather_body,
        out_type=jax.ShapeDtypeStruct((A_TOT, DIM), jnp.float32),
        mesh=mesh,
        scratch_types=[
            pltpu.VMEM((NCH, CH), jnp.int32),
            pltpu.VMEM((CH, DIM), jnp.float32),
            pltpu.VMEM((CH, DIM), jnp.float32),
            pltpu.SemaphoreType.DMA,
            pltpu.SemaphoreType.DMA,
            pltpu.SemaphoreType.DMA,
            pltpu.SemaphoreType.DMA,
        ],
    )
    return dispatch, gather


def kernel(x, Wr, br, We1, be1, We2, be2, We3, be3, Ws1, bs1, Ws2, bs2, Ws3,
           bs3):
    xf = x.reshape(T, DIM)
    logits = xf @ Wr + br

    er, rr, vv, cnts = pl.pallas_call(
        _router_body,
        grid=(T // TB,),
        in_specs=[
            pl.BlockSpec((TB, E), lambda i: (i, 0)),
        ],
        out_specs=[
            pl.BlockSpec((TB, 2), lambda i: (i, 0)),
            pl.BlockSpec((TB, 2), lambda i: (i, 0)),
            pl.BlockSpec((TB, 2), lambda i: (i, 0)),
            pl.BlockSpec((8, 8), lambda i: (0, 0)),
        ],
        out_shape=[
            jax.ShapeDtypeStruct((T, 2), jnp.int32),
            jax.ShapeDtypeStruct((T, 2), jnp.int32),
            jax.ShapeDtypeStruct((T, 2), jnp.float32),
            jax.ShapeDtypeStruct((8, 8), jnp.float32),
        ],
    )(logits)

    z = pl.pallas_call(
        _shared_body,
        grid=(T // TBS,),
        in_specs=[
            pl.BlockSpec((TBS, DIM), lambda i: (i, 0)),
            pl.BlockSpec((DIM, 2 * INTER), lambda i: (0, 0)),
            pl.BlockSpec((1, 2 * INTER), lambda i: (0, 0)),
            pl.BlockSpec((DIM, 2 * INTER), lambda i: (0, 0)),
            pl.BlockSpec((1, 2 * INTER), lambda i: (0, 0)),
            pl.BlockSpec((2 * INTER, DIM), lambda i: (0, 0)),
            pl.BlockSpec((1, DIM), lambda i: (0, 0)),
        ],
        out_specs=pl.BlockSpec((TBS, DIM), lambda i: (i, 0)),
        out_shape=jax.ShapeDtypeStruct((T, DIM), jnp.float32),
    )(xf, Ws1, bs1.reshape(1, -1), Ws2, bs2.reshape(1, -1), Ws3,
      bs3.reshape(1, -1))

    counts = cnts[0, :E].astype(jnp.int32)
    nblk = (counts + BLK - 1) // BLK
    blk_cum = jnp.cumsum(nblk)
    starts = (blk_cum - nblk) * BLK
    pos = jnp.take(starts, er.reshape(-1)) + rr.reshape(-1)
    blk_expert = jnp.minimum(
        jnp.searchsorted(blk_cum, jnp.arange(NB, dtype=jnp.int32),
                         side="right"),
        E - 1).astype(jnp.int32)

    pos2d = pos.reshape(NWK * NCH, CH)
    src2d = (jnp.arange(A_TOT, dtype=jnp.int32) // K).reshape(NWK * NCH, CH)

    sc_dispatch, sc_gather = _make_sc_kernels()
    x_g = sc_dispatch(xf, src2d, pos2d)

    og = pl.pallas_call(
        _gmm_body,
        grid_spec=pltpu.PrefetchScalarGridSpec(
            num_scalar_prefetch=1,
            grid=(NB,),
            in_specs=[
                pl.BlockSpec((BLK, DIM), lambda i, be: (i, 0)),
                pl.BlockSpec((1, DIM, INTER), lambda i, be: (be[i], 0, 0)),
                pl.BlockSpec((1, 1, INTER), lambda i, be: (be[i], 0, 0)),
                pl.BlockSpec((1, DIM, INTER), lambda i, be: (be[i], 0, 0)),
                pl.BlockSpec((1, 1, INTER), lambda i, be: (be[i], 0, 0)),
                pl.BlockSpec((1, INTER, DIM), lambda i, be: (be[i], 0, 0)),
                pl.BlockSpec((1, 1, DIM), lambda i, be: (be[i], 0, 0)),
            ],
            out_specs=pl.BlockSpec((BLK, DIM), lambda i, be: (i, 0)),
        ),
        out_shape=jax.ShapeDtypeStruct((G, DIM), jnp.float32),
    )(blk_expert, x_g, We1, be1.reshape(E, 1, INTER), We2,
      be2.reshape(E, 1, INTER), We3, be3.reshape(E, 1, DIM))

    og_tok = sc_gather(og, pos2d).reshape(T, 2 * DIM)

    y = pl.pallas_call(
        _combine_body,
        grid=(T // TBS,),
        in_specs=[
            pl.BlockSpec((TBS, DIM), lambda i: (i, 0)),
            pl.BlockSpec((TBS, 2 * DIM), lambda i: (i, 0)),
            pl.BlockSpec((TBS, 2), lambda i: (i, 0)),
        ],
        out_specs=pl.BlockSpec((TBS, DIM), lambda i: (i, 0)),
        out_shape=jax.ShapeDtypeStruct((T, DIM), jnp.float32),
    )(z, og_tok, vv)

    return y.reshape(x.shape)


# speedup vs baseline: 1.0765x; 1.0765x over previous
import jax
import jax.numpy as jnp
from jax import lax
from jax.experimental import pallas as pl
from jax.experimental.pallas import tpu as pltpu
from jax.experimental.pallas import tpu_sc as plsc

B, S = 2, 2048
DIM = 1024
INTER = 512
E = 8
K = 2
NG = 4
T = B * S
A_TOT = T * K
BLK = 256
G = A_TOT + E * BLK
NB = G // BLK
TB = 512
TBS = 512
NEG = -1e30

NC = 2
NS = 16
NWK = NC * NS
A_PER_W = A_TOT // NWK
CH = 64
NCH = A_PER_W // CH


def _router_body(lg_ref, er_ref, rr_ref, vv_ref, cnt_ref):
    i = pl.program_id(0)

    @pl.when(i == 0)
    def _():
        cnt_ref[...] = jnp.zeros((8, 8), jnp.float32)

    lane = lax.broadcasted_iota(jnp.int32, (TB, E), 1)
    lg = lg_ref[...]
    m = jnp.max(lg, axis=1, keepdims=True)
    ex = jnp.exp(lg - m)
    s = ex / jnp.sum(ex, axis=1, keepdims=True)

    row = lax.broadcasted_iota(jnp.int32, (E, E), 0)
    col = lax.broadcasted_iota(jnp.int32, (E, E), 1)
    gm = jnp.where((col < NG) & (row // 2 == col), 1.0, 0.0)
    gmt = jnp.where((row < NG) & (col // 2 == row), 1.0, 0.0)
    gs = jnp.dot(s, gm, preferred_element_type=jnp.float32,
                 precision=lax.Precision.HIGHEST)
    gsm = jnp.where(lane < NG, gs, NEG)

    g1 = jnp.max(gsm, axis=1, keepdims=True)
    i1 = jnp.min(jnp.where(gsm == g1, lane, 128), axis=1, keepdims=True)
    gsm2 = jnp.where(lane == i1, NEG, gsm)
    g2 = jnp.max(gsm2, axis=1, keepdims=True)
    i2 = jnp.min(jnp.where(gsm2 == g2, lane, 128), axis=1, keepdims=True)
    selg = jnp.where((lane == i1) | (lane == i2), 1.0, 0.0)
    sele = jnp.dot(selg, gmt, preferred_element_type=jnp.float32,
                   precision=lax.Precision.HIGHEST)

    sp = jnp.where(sele > 0.5, s, NEG)
    v1 = jnp.max(sp, axis=1, keepdims=True)
    e1 = jnp.min(jnp.where(sp == v1, lane, 128), axis=1, keepdims=True)
    sp2 = jnp.where(lane == e1, NEG, sp)
    v2 = jnp.max(sp2, axis=1, keepdims=True)
    e2 = jnp.min(jnp.where(sp2 == v2, lane, 128), axis=1, keepdims=True)

    oh1 = jnp.where(lane == e1, 1.0, 0.0)
    oh2 = jnp.where(lane == e2, 1.0, 0.0)
    h = oh1 + oh2
    rr = lax.broadcasted_iota(jnp.int32, (TB, TB), 0)
    cc = lax.broadcasted_iota(jnp.int32, (TB, TB), 1)
    lf = jnp.where(cc < rr, 1.0, 0.0)
    p = jnp.dot(lf, h, preferred_element_type=jnp.float32)
    cb = cnt_ref[0:1, :]
    r1 = jnp.sum((p + cb) * oh1, axis=1, keepdims=True)
    r2 = jnp.sum((p + cb) * oh2, axis=1, keepdims=True)
    cnt_ref[0:1, :] = cb + jnp.sum(h, axis=0, keepdims=True)

    er_ref[...] = jnp.concatenate([e1, e2], axis=1)
    rr_ref[...] = jnp.concatenate([r1.astype(jnp.int32),
                                   r2.astype(jnp.int32)], axis=1)
    vv_ref[...] = jnp.concatenate([v1, v2], axis=1)


def _gmm_body(be_ref, x_ref, w1_ref, b1_ref, w2_ref, b2_ref, w3_ref, b3_ref,
              o_ref):
    a = x_ref[...]
    h1 = jnp.dot(a, w1_ref[0], preferred_element_type=jnp.float32) + b1_ref[0]
    h2 = jnp.dot(a, w2_ref[0], preferred_element_type=jnp.float32) + b2_ref[0]
    h = h1 * h2
    hs = h * jax.nn.sigmoid(h)
    o_ref[...] = jnp.dot(hs, w3_ref[0],
                         preferred_element_type=jnp.float32) + b3_ref[0]


def _shared_body(x_ref, ws1_ref, bs1_ref, ws2_ref, bs2_ref, ws3_ref, bs3_ref,
                 z_ref):
    a = x_ref[...]
    h1 = jnp.dot(a, ws1_ref[...], preferred_element_type=jnp.float32) + bs1_ref[...]
    h2 = jnp.dot(a, ws2_ref[...], preferred_element_type=jnp.float32) + bs2_ref[...]
    h = h1 * h2
    hs = h * jax.nn.sigmoid(h)
    z_ref[...] = jnp.dot(hs, ws3_ref[...],
                         preferred_element_type=jnp.float32) + bs3_ref[...]


def _combine_body(z_ref, og_ref, vv_ref, y_ref):
    vv = vv_ref[...]
    v1 = vv[:, 0:1]
    v2 = vv[:, 1:2]
    og = og_ref[...]
    y_ref[...] = z_ref[...] + v1 * og[:, :DIM] + v2 * og[:, DIM:]


def _sc_dispatch_body(x_hbm, src_hbm, pos_hbm, xg_hbm, src_v, pos_v,
                      rows_a, sga, ssa):
    wid = lax.axis_index("s") * NC + lax.axis_index("c")
    base = wid * NCH
    pltpu.sync_copy(src_hbm.at[pl.ds(base, NCH)], src_v)
    pltpu.sync_copy(pos_hbm.at[pl.ds(base, NCH)], pos_v)
    for j in range(NCH):
        pltpu.async_copy(x_hbm.at[src_v.at[j]], rows_a, sga).wait()
        pltpu.async_copy(rows_a, xg_hbm.at[pos_v.at[j]], ssa).wait()


def _sc_gather_body(og_hbm, pos_hbm, out_hbm, pos_v, rows_a, sga, ssa):
    wid = lax.axis_index("s") * NC + lax.axis_index("c")
    base = wid * NCH
    abase = wid * A_PER_W
    pltpu.sync_copy(pos_hbm.at[pl.ds(base, NCH)], pos_v)
    for j in range(NCH):
        pltpu.async_copy(og_hbm.at[pos_v.at[j]], rows_a, sga).wait()
        pltpu.async_copy(rows_a, out_hbm.at[pl.ds(abase + j * CH, CH)],
                         ssa).wait()


def _make_sc_kernels():
    mesh = plsc.VectorSubcoreMesh(core_axis_name="c", subcore_axis_name="s",
                                  num_cores=NC, num_subcores=NS)
    dispatch = pl.kernel(
        _sc_dispatch_body,
        out_type=jax.ShapeDtypeStruct((G, DIM), jnp.float32),
        mesh=mesh,
        scratch_types=[
            pltpu.VMEM((NCH, CH), jnp.int32),
            pltpu.VMEM((NCH, CH), jnp.int32),
            pltpu.VMEM((CH, DIM), jnp.float32),
            pltpu.SemaphoreType.DMA,
            pltpu.SemaphoreType.DMA,
        ],
    )
    gather = pl.kernel(
        _sc_gather_body,
        out_type=jax.ShapeDtypeStruct((A_TOT, DIM), jnp.float32),
        mesh=mesh,
        scratch_types=[
            pltpu.VMEM((NCH, CH), jnp.int32),
            pltpu.VMEM((CH, DIM), jnp.float32),
            pltpu.SemaphoreType.DMA,
            pltpu.SemaphoreType.DMA,
        ],
    )
    return dispatch, gather


def kernel(x, Wr, br, We1, be1, We2, be2, We3, be3, Ws1, bs1, Ws2, bs2, Ws3,
           bs3):
    xf = x.reshape(T, DIM)
    logits = xf @ Wr + br

    er, rr, vv, cnts = pl.pallas_call(
        _router_body,
        grid=(T // TB,),
        in_specs=[
            pl.BlockSpec((TB, E), lambda i: (i, 0)),
        ],
        out_specs=[
            pl.BlockSpec((TB, 2), lambda i: (i, 0)),
            pl.BlockSpec((TB, 2), lambda i: (i, 0)),
            pl.BlockSpec((TB, 2), lambda i: (i, 0)),
            pl.BlockSpec((8, 8), lambda i: (0, 0)),
        ],
        out_shape=[
            jax.ShapeDtypeStruct((T, 2), jnp.int32),
            jax.ShapeDtypeStruct((T, 2), jnp.int32),
            jax.ShapeDtypeStruct((T, 2), jnp.float32),
            jax.ShapeDtypeStruct((8, 8), jnp.float32),
        ],
    )(logits)

    z = pl.pallas_call(
        _shared_body,
        grid=(T // TBS,),
        in_specs=[
            pl.BlockSpec((TBS, DIM), lambda i: (i, 0)),
            pl.BlockSpec((DIM, 2 * INTER), lambda i: (0, 0)),
            pl.BlockSpec((1, 2 * INTER), lambda i: (0, 0)),
            pl.BlockSpec((DIM, 2 * INTER), lambda i: (0, 0)),
            pl.BlockSpec((1, 2 * INTER), lambda i: (0, 0)),
            pl.BlockSpec((2 * INTER, DIM), lambda i: (0, 0)),
            pl.BlockSpec((1, DIM), lambda i: (0, 0)),
        ],
        out_specs=pl.BlockSpec((TBS, DIM), lambda i: (i, 0)),
        out_shape=jax.ShapeDtypeStruct((T, DIM), jnp.float32),
    )(xf, Ws1, bs1.reshape(1, -1), Ws2, bs2.reshape(1, -1), Ws3,
      bs3.reshape(1, -1))

    counts = cnts[0, :E].astype(jnp.int32)
    nblk = (counts + BLK - 1) // BLK
    blk_cum = jnp.cumsum(nblk)
    starts = (blk_cum - nblk) * BLK
    pos = jnp.take(starts, er.reshape(-1)) + rr.reshape(-1)
    blk_expert = jnp.minimum(
        jnp.searchsorted(blk_cum, jnp.arange(NB, dtype=jnp.int32),
                         side="right"),
        E - 1).astype(jnp.int32)

    pos2d = pos.reshape(NWK * NCH, CH)
    src2d = (jnp.arange(A_TOT, dtype=jnp.int32) // K).reshape(NWK * NCH, CH)

    sc_dispatch, sc_gather = _make_sc_kernels()
    x_g = sc_dispatch(xf, src2d, pos2d)

    og = pl.pallas_call(
        _gmm_body,
        grid_spec=pltpu.PrefetchScalarGridSpec(
            num_scalar_prefetch=1,
            grid=(NB,),
            in_specs=[
                pl.BlockSpec((BLK, DIM), lambda i, be: (i, 0)),
                pl.BlockSpec((1, DIM, INTER), lambda i, be: (be[i], 0, 0)),
                pl.BlockSpec((1, 1, INTER), lambda i, be: (be[i], 0, 0)),
                pl.BlockSpec((1, DIM, INTER), lambda i, be: (be[i], 0, 0)),
                pl.BlockSpec((1, 1, INTER), lambda i, be: (be[i], 0, 0)),
                pl.BlockSpec((1, INTER, DIM), lambda i, be: (be[i], 0, 0)),
                pl.BlockSpec((1, 1, DIM), lambda i, be: (be[i], 0, 0)),
            ],
            out_specs=pl.BlockSpec((BLK, DIM), lambda i, be: (i, 0)),
        ),
        out_shape=jax.ShapeDtypeStruct((G, DIM), jnp.float32),
    )(blk_expert, x_g, We1, be1.reshape(E, 1, INTER), We2,
      be2.reshape(E, 1, INTER), We3, be3.reshape(E, 1, DIM))

    og_tok = sc_gather(og, pos2d).reshape(T, 2 * DIM)

    y = pl.pallas_call(
        _combine_body,
        grid=(T // TBS,),
        in_specs=[
            pl.BlockSpec((TBS, DIM), lambda i: (i, 0)),
            pl.BlockSpec((TBS, 2 * DIM), lambda i: (i, 0)),
            pl.BlockSpec((TBS, 2), lambda i: (i, 0)),
        ],
        out_specs=pl.BlockSpec((TBS, DIM), lambda i: (i, 0)),
        out_shape=jax.ShapeDtypeStruct((T, DIM), jnp.float32),
    )(z, og_tok, vv)

    return y.reshape(x.shape)
